# baseline (speedup 1.0000x reference)
import functools

import jax
import jax.numpy as jnp
from jax import lax
from jax.experimental import pallas as pl
from jax.experimental.pallas import tpu as pltpu
from jax.experimental.pallas import tpu_sc as plsc

_NW = 32
_CHUNK = 32
_NBUF = 3


def _body(emb_ref, anchor_ref, out_ref, buf, in_sem, out_sem):
    R, D = out_ref.shape
    S = 4096
    per_w = R // _NW
    nchunks = per_w // _CHUNK
    wid = lax.axis_index("s") * 2 + lax.axis_index("c")
    base = wid * per_w
    owns_row0 = (base % S) == 0

    def in_cp(i, s):
        return pltpu.make_async_copy(
            emb_ref.at[pl.ds(base + i * _CHUNK, _CHUNK), :],
            buf.at[s],
            in_sem.at[s],
        )

    def out_cp(i, s):
        return pltpu.make_async_copy(
            buf.at[s],
            out_ref.at[pl.ds(base + i * _CHUNK, _CHUNK), :],
            out_sem.at[s],
        )

    def emit(i):
        s = i % _NBUF
        if i >= _NBUF:
            out_cp(i - _NBUF, s).wait()
        in_cp(i, s).start()
        if i >= 1:
            sp = (i - 1) % _NBUF
            in_cp(i - 1, sp).wait()
            if i - 1 == 0:
                @pl.when(owns_row0)
                def _():
                    pltpu.sync_copy(anchor_ref.at[0, :], buf.at[sp, 0, :])
            out_cp(i - 1, sp).start()

    for i in range(nchunks):
        emit(i)
    last = nchunks - 1
    s = last % _NBUF
    in_cp(last, s).wait()
    out_cp(last, s).start()
    for i in range(max(0, nchunks - _NBUF), nchunks):
        out_cp(i, i % _NBUF).wait()


@jax.jit
def _run(token_embeddings, style_anchor):
    B, S, D = token_embeddings.shape
    flat = token_embeddings.reshape(B * S, D)
    k = pl.kernel(
        _body,
        out_type=jax.ShapeDtypeStruct((B * S, D), token_embeddings.dtype),
        mesh=plsc.VectorSubcoreMesh(core_axis_name="c", subcore_axis_name="s"),
        scratch_types=[
            pltpu.VMEM((_NBUF, _CHUNK, D), jnp.float32),
            pltpu.SemaphoreType.DMA((_NBUF,)),
            pltpu.SemaphoreType.DMA((_NBUF,)),
        ],
    )
    out = k(flat, style_anchor)
    return out.reshape(B, S, D)


def kernel(token_embeddings, style_anchor):
    return _run(token_embeddings, style_anchor)


# speedup vs baseline: 1.5790x; 1.5790x over previous
import jax
import jax.numpy as jnp
from jax.experimental import pallas as pl
from jax.experimental.pallas import tpu as pltpu

_CHUNK = 1024
_NBUF = 4


def _body(emb_ref, anchor_ref, out_ref, buf, in_sem, out_sem):
    R, D = out_ref.shape
    S = 4096
    nchunks = R // _CHUNK

    def start_in(i):
        pltpu.make_async_copy(
            emb_ref.at[pl.ds(i * _CHUNK, _CHUNK), :],
            buf.at[i % _NBUF],
            in_sem.at[i % _NBUF],
        ).start()

    for i in range(min(_NBUF, nchunks)):
        start_in(i)
    for i in range(nchunks):
        pltpu.make_async_copy(
            emb_ref.at[pl.ds(i * _CHUNK, _CHUNK), :],
            buf.at[i % _NBUF],
            in_sem.at[i % _NBUF],
        ).wait()
        if (i * _CHUNK) % S == 0:
            buf[i % _NBUF, 0, :] = anchor_ref[0, :]
        out_cp = pltpu.make_async_copy(
            buf.at[i % _NBUF],
            out_ref.at[pl.ds(i * _CHUNK, _CHUNK), :],
            out_sem.at[i % _NBUF],
        )
        out_cp.start()
        if i + _NBUF < nchunks:
            out_cp.wait()
            start_in(i + _NBUF)
    for i in range(max(0, nchunks - _NBUF), nchunks):
        pltpu.make_async_copy(
            buf.at[i % _NBUF],
            out_ref.at[pl.ds(i * _CHUNK, _CHUNK), :],
            out_sem.at[i % _NBUF],
        ).wait()


@jax.jit
def _run(token_embeddings, style_anchor):
    B, S, D = token_embeddings.shape
    flat = token_embeddings.reshape(B * S, D)
    out = pl.pallas_call(
        _body,
        in_specs=[
            pl.BlockSpec(memory_space=pltpu.MemorySpace.HBM),
            pl.BlockSpec(memory_space=pltpu.MemorySpace.VMEM),
        ],
        out_specs=pl.BlockSpec(memory_space=pltpu.MemorySpace.HBM),
        out_shape=jax.ShapeDtypeStruct((B * S, D), token_embeddings.dtype),
        scratch_shapes=[
            pltpu.VMEM((_NBUF, _CHUNK, D), jnp.float32),
            pltpu.SemaphoreType.DMA((_NBUF,)),
            pltpu.SemaphoreType.DMA((_NBUF,)),
        ],
    )(flat, style_anchor)
    return out.reshape(B, S, D)


def kernel(token_embeddings, style_anchor):
    return _run(token_embeddings, style_anchor)


# speedup vs baseline: 1.6172x; 1.0242x over previous
import jax
import jax.numpy as jnp
from jax.experimental import pallas as pl
from jax.experimental.pallas import tpu as pltpu

_CHUNK = 1024
_NBUF = 8


def _body(emb_ref, anchor_ref, out_ref, buf, in_sem, out_sem):
    R, D = out_ref.shape
    S = 4096
    nchunks = R // _CHUNK

    def start_in(i):
        pltpu.make_async_copy(
            emb_ref.at[pl.ds(i * _CHUNK, _CHUNK), :],
            buf.at[i % _NBUF],
            in_sem.at[i % _NBUF],
        ).start()

    for i in range(min(_NBUF, nchunks)):
        start_in(i)
    for i in range(nchunks):
        pltpu.make_async_copy(
            emb_ref.at[pl.ds(i * _CHUNK, _CHUNK), :],
            buf.at[i % _NBUF],
            in_sem.at[i % _NBUF],
        ).wait()
        if (i * _CHUNK) % S == 0:
            buf[i % _NBUF, 0, :] = anchor_ref[0, :]
        out_cp = pltpu.make_async_copy(
            buf.at[i % _NBUF],
            out_ref.at[pl.ds(i * _CHUNK, _CHUNK), :],
            out_sem.at[i % _NBUF],
        )
        out_cp.start()
        if i + _NBUF < nchunks:
            out_cp.wait()
            start_in(i + _NBUF)
    for i in range(max(0, nchunks - _NBUF), nchunks):
        pltpu.make_async_copy(
            buf.at[i % _NBUF],
            out_ref.at[pl.ds(i * _CHUNK, _CHUNK), :],
            out_sem.at[i % _NBUF],
        ).wait()


@jax.jit
def _run(token_embeddings, style_anchor):
    B, S, D = token_embeddings.shape
    flat = token_embeddings.reshape(B * S, D)
    out = pl.pallas_call(
        _body,
        in_specs=[
            pl.BlockSpec(memory_space=pltpu.MemorySpace.HBM),
            pl.BlockSpec(memory_space=pltpu.MemorySpace.VMEM),
        ],
        out_specs=pl.BlockSpec(memory_space=pltpu.MemorySpace.HBM),
        out_shape=jax.ShapeDtypeStruct((B * S, D), token_embeddings.dtype),
        scratch_shapes=[
            pltpu.VMEM((_NBUF, _CHUNK, D), jnp.float32),
            pltpu.SemaphoreType.DMA((_NBUF,)),
            pltpu.SemaphoreType.DMA((_NBUF,)),
        ],
    )(flat, style_anchor)
    return out.reshape(B, S, D)


def kernel(token_embeddings, style_anchor):
    return _run(token_embeddings, style_anchor)
